# Initial kernel scaffold; baseline (speedup 1.0000x reference)
import jax
import jax.numpy as jnp
from jax.experimental import pallas as pl


def kernel(features, A, weight, bias):
    raise NotImplementedError("write your pallas kernel here")


# speedup vs baseline: 1.0059x; 1.0059x over previous
import jax
import jax.numpy as jnp
from jax.experimental import pallas as pl
from jax.experimental.pallas import tpu as pltpu

_N = 10000
_D = 128
_BM = 200


def _graphconv_body(a_ref, f_ref, fm_ref, w_ref, b_ref, o_ref):
    agg = jnp.dot(a_ref[...], f_ref[...], preferred_element_type=jnp.float32)
    out = jnp.dot(fm_ref[...], w_ref[:_D, :], preferred_element_type=jnp.float32)
    out += jnp.dot(agg, w_ref[_D:, :], preferred_element_type=jnp.float32)
    o_ref[...] = jnp.maximum(out + b_ref[...], 0.0)


def kernel(features, A, weight, bias):
    grid = (_N // _BM,)
    return pl.pallas_call(
        _graphconv_body,
        grid=grid,
        in_specs=[
            pl.BlockSpec((_BM, _N), lambda m: (m, 0)),
            pl.BlockSpec((_N, _D), lambda m: (0, 0)),
            pl.BlockSpec((_BM, _D), lambda m: (m, 0)),
            pl.BlockSpec((2 * _D, _D), lambda m: (0, 0)),
            pl.BlockSpec((1, _D), lambda m: (0, 0)),
        ],
        out_specs=pl.BlockSpec((_BM, _D), lambda m: (m, 0)),
        out_shape=jax.ShapeDtypeStruct((_N, _D), jnp.float32),
        compiler_params=pltpu.CompilerParams(
            dimension_semantics=("arbitrary",),
        ),
    )(A, features, features, weight, bias.reshape(1, _D))


# speedup vs baseline: 1.0256x; 1.0196x over previous
import jax
import jax.numpy as jnp
from jax.experimental import pallas as pl
from jax.experimental.pallas import tpu as pltpu

_N = 10000
_D = 128
_BM = 400


def _graphconv_body(a_ref, f_ref, fm_ref, w_ref, b_ref, o_ref):
    agg = jnp.dot(a_ref[...], f_ref[...], preferred_element_type=jnp.float32)
    out = jnp.dot(fm_ref[...], w_ref[:_D, :], preferred_element_type=jnp.float32)
    out += jnp.dot(agg, w_ref[_D:, :], preferred_element_type=jnp.float32)
    o_ref[...] = jnp.maximum(out + b_ref[...], 0.0)


def kernel(features, A, weight, bias):
    grid = (_N // _BM,)
    return pl.pallas_call(
        _graphconv_body,
        grid=grid,
        in_specs=[
            pl.BlockSpec((_BM, _N), lambda m: (m, 0)),
            pl.BlockSpec((_N, _D), lambda m: (0, 0)),
            pl.BlockSpec((_BM, _D), lambda m: (m, 0)),
            pl.BlockSpec((2 * _D, _D), lambda m: (0, 0)),
            pl.BlockSpec((1, _D), lambda m: (0, 0)),
        ],
        out_specs=pl.BlockSpec((_BM, _D), lambda m: (m, 0)),
        out_shape=jax.ShapeDtypeStruct((_N, _D), jnp.float32),
        compiler_params=pltpu.CompilerParams(
            dimension_semantics=("arbitrary",),
        ),
    )(A, features, features, weight, bias.reshape(1, _D))


# speedup vs baseline: 1.0295x; 1.0038x over previous
import jax
import jax.numpy as jnp
from jax.experimental import pallas as pl
from jax.experimental.pallas import tpu as pltpu

_N = 10000
_D = 128
_BM = 400


def _graphconv_body(a_ref, f_ref, fm_ref, w_ref, b_ref, o_ref):
    agg = jnp.dot(a_ref[...], f_ref[...], preferred_element_type=jnp.float32)
    out = jnp.dot(fm_ref[...], w_ref[:_D, :], preferred_element_type=jnp.float32)
    out += jnp.dot(agg, w_ref[_D:, :], preferred_element_type=jnp.float32)
    o_ref[...] = jnp.maximum(out + b_ref[...], 0.0)


def kernel(features, A, weight, bias):
    grid = (_N // _BM,)
    return pl.pallas_call(
        _graphconv_body,
        grid=grid,
        in_specs=[
            pl.BlockSpec((_BM, _N), lambda m: (m, 0)),
            pl.BlockSpec((_N, _D), lambda m: (0, 0)),
            pl.BlockSpec((_BM, _D), lambda m: (m, 0)),
            pl.BlockSpec((2 * _D, _D), lambda m: (0, 0)),
            pl.BlockSpec((1, _D), lambda m: (0, 0)),
        ],
        out_specs=pl.BlockSpec((_BM, _D), lambda m: (m, 0)),
        out_shape=jax.ShapeDtypeStruct((_N, _D), jnp.float32),
        compiler_params=pltpu.CompilerParams(
            dimension_semantics=("parallel",),
        ),
    )(A, features, features, weight, bias.reshape(1, _D))


# speedup vs baseline: 1.0661x; 1.0356x over previous
import jax
import jax.numpy as jnp
from jax.experimental import pallas as pl
from jax.experimental.pallas import tpu as pltpu

_N = 10000
_D = 128
_BM = 400


def _graphconv_body(a_ref, f_ref, w_ref, b_ref, o_ref):
    m = pl.program_id(0)
    agg = jnp.dot(a_ref[...], f_ref[...], preferred_element_type=jnp.float32)
    fm = f_ref[pl.ds(m * _BM, _BM), :]
    out = jnp.dot(fm, w_ref[:_D, :], preferred_element_type=jnp.float32)
    out += jnp.dot(agg, w_ref[_D:, :], preferred_element_type=jnp.float32)
    o_ref[...] = jnp.maximum(out + b_ref[...], 0.0)


def kernel(features, A, weight, bias):
    grid = (_N // _BM,)
    return pl.pallas_call(
        _graphconv_body,
        grid=grid,
        in_specs=[
            pl.BlockSpec((_BM, _N), lambda m: (m, 0)),
            pl.BlockSpec((_N, _D), lambda m: (0, 0)),
            pl.BlockSpec((2 * _D, _D), lambda m: (0, 0)),
            pl.BlockSpec((1, _D), lambda m: (0, 0)),
        ],
        out_specs=pl.BlockSpec((_BM, _D), lambda m: (m, 0)),
        out_shape=jax.ShapeDtypeStruct((_N, _D), jnp.float32),
        compiler_params=pltpu.CompilerParams(
            dimension_semantics=("parallel",),
        ),
    )(A, features, weight, bias.reshape(1, _D))
